# baseline (speedup 1.0000x reference)
import functools

import jax
import jax.numpy as jnp
from jax import lax
from jax.experimental import pallas as pl
from jax.experimental.pallas import tpu as pltpu
from jax.experimental.pallas import tpu_sc as plsc

NUM_EMBEDDINGS = 1024
EMBEDDING_DIM = 256
SAMPLE_TOKENS = 8
CODE_DIM = 32
COMMITMENT_COST = 0.25

BLOCK_T = 512

_SC_INFO = plsc.get_sparse_core_info()
_NC, _NS, _L = _SC_INFO.num_cores, _SC_INFO.num_subcores, _SC_INFO.num_lanes
_NW = _NC * _NS


def _tc_body(x_ref, wt2_ref, w_ref, wsq_ref, colf_ref, q_ref, idx_ref,
             sq_ref, acc_ref, *, n_blocks):
    step = pl.program_id(0)
    wt2 = wt2_ref[...]
    w = w_ref[...]
    wsq = wsq_ref[...]
    colf = colf_ref[...]

    blk_sq = None
    idx_cols = []
    for s in range(SAMPLE_TOKENS):
        xs = x_ref[:, s * CODE_DIM:(s + 1) * CODE_DIM]
        scores2 = jnp.dot(xs, wt2, preferred_element_type=jnp.float32)
        xsq = jnp.sum(xs * xs, axis=1, keepdims=True)
        dist = (xsq + wsq) - scores2
        dmin = jnp.min(dist, axis=1, keepdims=True)
        idxf = jnp.min(jnp.where(dist == dmin, colf, float(NUM_EMBEDDINGS)),
                       axis=1, keepdims=True)
        idx_cols.append(idxf.astype(jnp.int32))
        onehot = (colf == idxf).astype(jnp.float32)
        q = jnp.dot(onehot, w, preferred_element_type=jnp.float32)
        q_ref[:, s * CODE_DIM:(s + 1) * CODE_DIM] = q
        sq_s = jnp.sum(dmin)
        blk_sq = sq_s if blk_sq is None else blk_sq + sq_s

    idx_ref[...] = jnp.concatenate(idx_cols, axis=1)

    @pl.when(step == 0)
    def _init():
        acc_ref[0, 0] = blk_sq

    @pl.when(step > 0)
    def _acc():
        acc_ref[0, 0] += blk_sq

    @pl.when(step == n_blocks - 1)
    def _finalize():
        sq_ref[...] = jnp.reshape(acc_ref[0, 0], (1, 1))


def _make_sc_kernel(n_rows):
    bpw = n_rows // _NW
    mesh = plsc.VectorSubcoreMesh(core_axis_name="c", subcore_axis_name="s")

    @functools.partial(
        pl.kernel, mesh=mesh,
        out_type=[
            jax.ShapeDtypeStruct((_NC, NUM_EMBEDDINGS), jnp.float32),
        ],
        scratch_types=[
            pltpu.VMEM((bpw,), jnp.int32),
            pltpu.VMEM((bpw,), jnp.float32),
            pltpu.VMEM((NUM_EMBEDDINGS,), jnp.float32),
            pltpu.VMEM_SHARED((NUM_EMBEDDINGS,), jnp.float32),
        ],
        compiler_params=pltpu.CompilerParams(use_tc_tiling_on_sc=False),
    )
    def sc_hist(idx_hbm, cnt_hbm, idx_v, ones_v, bounce_v, cnt_sh):
        cid = lax.axis_index("c")
        sid = lax.axis_index("s")
        wid = sid * _NC + cid
        base = wid * bpw
        pltpu.sync_copy(idx_hbm.at[pl.ds(base, bpw)], idx_v)

        zero = jnp.zeros((_L,), jnp.float32)
        one = jnp.ones((_L,), jnp.float32)
        for i in range(bpw // _L):
            ones_v[pl.ds(i * _L, _L)] = one

        @pl.when(sid == 0)
        def _init():
            for i in range(NUM_EMBEDDINGS // _L):
                bounce_v[pl.ds(i * _L, _L)] = zero
            pltpu.sync_copy(bounce_v, cnt_sh)

        plsc.subcore_barrier()
        pltpu.sync_copy(ones_v, cnt_sh.at[idx_v], add=True)
        plsc.subcore_barrier()

        @pl.when(sid == 0)
        def _emit():
            pltpu.sync_copy(cnt_sh, bounce_v)
            pltpu.sync_copy(bounce_v, cnt_hbm.at[cid])

    return sc_hist


def _perp_body(cnt_ref, perp_ref, *, n_rows):
    c = cnt_ref[...]
    probs = jnp.sum(c, axis=0, keepdims=True) / n_rows
    ent = jnp.sum(probs * jnp.log(probs + 1e-10), axis=1, keepdims=True)
    perp_ref[...] = jnp.exp(-ent)


def kernel(inputs, W):
    input_shape = inputs.shape
    x2 = inputs.reshape(-1, EMBEDDING_DIM)
    n_tok = x2.shape[0]
    n_rows = n_tok * SAMPLE_TOKENS

    HALVES = 1
    tok_h = n_tok // HALVES
    rows_h = n_rows // HALVES
    n_blocks = tok_h // BLOCK_T

    wt2 = 2.0 * W.T
    wsq = jnp.sum(W ** 2, axis=1)[None, :]
    colf = jnp.arange(NUM_EMBEDDINGS, dtype=jnp.float32)[None, :]

    sc_call = _make_sc_kernel(rows_h)

    def tc_call(h):
        off = h * (tok_h // BLOCK_T)
        return pl.pallas_call(
            functools.partial(_tc_body, n_blocks=n_blocks),
            grid=(n_blocks,),
            in_specs=[
                pl.BlockSpec((BLOCK_T, EMBEDDING_DIM), lambda i: (i + off, 0)),
                pl.BlockSpec((CODE_DIM, NUM_EMBEDDINGS), lambda i: (0, 0)),
                pl.BlockSpec((NUM_EMBEDDINGS, CODE_DIM), lambda i: (0, 0)),
                pl.BlockSpec((1, NUM_EMBEDDINGS), lambda i: (0, 0)),
                pl.BlockSpec((1, NUM_EMBEDDINGS), lambda i: (0, 0)),
            ],
            out_specs=[
                pl.BlockSpec((BLOCK_T, EMBEDDING_DIM), lambda i: (i, 0)),
                pl.BlockSpec((BLOCK_T, SAMPLE_TOKENS), lambda i: (i, 0)),
                pl.BlockSpec((1, 1), lambda i: (0, 0)),
            ],
            out_shape=[
                jax.ShapeDtypeStruct((tok_h, EMBEDDING_DIM), jnp.float32),
                jax.ShapeDtypeStruct((tok_h, SAMPLE_TOKENS), jnp.int32),
                jax.ShapeDtypeStruct((1, 1), jnp.float32),
            ],
            scratch_shapes=[
                pltpu.SMEM((1, 1), jnp.float32),
            ],
        )(x2, wt2, W, wsq, colf)

    qs, cnts, sq_total = [], [], None
    for h in range(HALVES):
        q_h, idx_h, sq_h = tc_call(h)
        cnt_h, = sc_call(idx_h.reshape(-1))
        qs.append(q_h)
        cnts.append(cnt_h)
        sq_total = sq_h if sq_total is None else sq_total + sq_h

    loss = ((1.0 + COMMITMENT_COST) / (n_rows * CODE_DIM)) * sq_total[0, 0]

    cnt = jnp.concatenate(cnts, axis=0)
    perp = pl.pallas_call(
        functools.partial(_perp_body, n_rows=n_rows),
        in_specs=[pl.BlockSpec((HALVES * _NC, NUM_EMBEDDINGS),
                               lambda: (0, 0))],
        out_specs=pl.BlockSpec((1, 1), lambda: (0, 0)),
        out_shape=jax.ShapeDtypeStruct((1, 1), jnp.float32),
    )(cnt)

    q = jnp.concatenate(qs, axis=0)
    return (q.reshape(input_shape), loss, perp[0, 0])


# speedup vs baseline: 1.4548x; 1.4548x over previous
import functools

import jax
import jax.numpy as jnp
from jax import lax
from jax.experimental import pallas as pl
from jax.experimental.pallas import tpu as pltpu
from jax.experimental.pallas import tpu_sc as plsc

NUM_EMBEDDINGS = 1024
EMBEDDING_DIM = 256
SAMPLE_TOKENS = 8
CODE_DIM = 32
COMMITMENT_COST = 0.25

BLOCK_T = 1024

_SC_INFO = plsc.get_sparse_core_info()
_NC, _NS, _L = _SC_INFO.num_cores, _SC_INFO.num_subcores, _SC_INFO.num_lanes
_NW = _NC * _NS


def _tc_body(x_ref, wt2_ref, wsq_ref, colf_ref, idx_ref, sq_ref, acc_ref,
             *, n_blocks):
    step = pl.program_id(0)
    wt2 = wt2_ref[...]
    wsq = wsq_ref[...]
    colf = colf_ref[...]

    blk_sq = None
    idx_cols = []
    for s in range(SAMPLE_TOKENS):
        xs = x_ref[:, s * CODE_DIM:(s + 1) * CODE_DIM]
        scores2 = jnp.dot(xs, wt2, preferred_element_type=jnp.float32)
        xsq = jnp.sum(xs * xs, axis=1, keepdims=True)
        dist = (xsq + wsq) - scores2
        dmin = jnp.min(dist, axis=1, keepdims=True)
        idxf = jnp.min(jnp.where(dist == dmin, colf, float(NUM_EMBEDDINGS)),
                       axis=1, keepdims=True)
        idx_cols.append(idxf.astype(jnp.int32))
        sq_s = jnp.sum(dmin)
        blk_sq = sq_s if blk_sq is None else blk_sq + sq_s

    idx_ref[...] = jnp.concatenate(idx_cols, axis=1)

    @pl.when(step == 0)
    def _init():
        acc_ref[0, 0] = blk_sq

    @pl.when(step > 0)
    def _acc():
        acc_ref[0, 0] += blk_sq

    @pl.when(step == n_blocks - 1)
    def _finalize():
        sq_ref[...] = jnp.reshape(acc_ref[0, 0], (1, 1))


def _make_sc_kernel(n_rows):
    bpw = n_rows // _NW
    mesh = plsc.VectorSubcoreMesh(core_axis_name="c", subcore_axis_name="s")

    @functools.partial(
        pl.kernel, mesh=mesh,
        out_type=[
            jax.ShapeDtypeStruct((n_rows, CODE_DIM), jnp.float32),
            jax.ShapeDtypeStruct((_NC, NUM_EMBEDDINGS), jnp.float32),
        ],
        scratch_types=[
            pltpu.VMEM((bpw,), jnp.int32),
            pltpu.VMEM((bpw, CODE_DIM), jnp.float32),
            pltpu.VMEM((bpw,), jnp.float32),
            pltpu.VMEM((NUM_EMBEDDINGS,), jnp.float32),
            pltpu.VMEM_SHARED((NUM_EMBEDDINGS,), jnp.float32),
            pltpu.SemaphoreType.DMA,
        ],
        compiler_params=pltpu.CompilerParams(use_tc_tiling_on_sc=False),
    )
    def sc_gather_hist(table_hbm, idx_hbm, out_hbm, cnt_hbm, idx_v, rows_v,
                       ones_v, bounce_v, cnt_sh, sem):
        cid = lax.axis_index("c")
        sid = lax.axis_index("s")
        wid = sid * _NC + cid
        base = wid * bpw
        pltpu.sync_copy(idx_hbm.at[pl.ds(base, bpw)], idx_v)
        pltpu.async_copy(table_hbm.at[idx_v], rows_v, sem).wait()
        pltpu.sync_copy(rows_v, out_hbm.at[pl.ds(base, bpw)])

        zero = jnp.zeros((_L,), jnp.float32)
        one = jnp.ones((_L,), jnp.float32)
        for i in range(bpw // _L):
            ones_v[pl.ds(i * _L, _L)] = one

        @pl.when(sid == 0)
        def _init():
            for i in range(NUM_EMBEDDINGS // _L):
                bounce_v[pl.ds(i * _L, _L)] = zero
            pltpu.sync_copy(bounce_v, cnt_sh)

        plsc.subcore_barrier()
        pltpu.sync_copy(ones_v, cnt_sh.at[idx_v], add=True)
        plsc.subcore_barrier()

        @pl.when(sid == 0)
        def _emit():
            pltpu.sync_copy(cnt_sh, bounce_v)
            pltpu.sync_copy(bounce_v, cnt_hbm.at[cid])

    return sc_gather_hist


def _perp_body(cnt_ref, perp_ref, *, n_rows):
    c = cnt_ref[...]
    probs = jnp.sum(c, axis=0, keepdims=True) / n_rows
    ent = jnp.sum(probs * jnp.log(probs + 1e-10), axis=1, keepdims=True)
    perp_ref[...] = jnp.exp(-ent)


def kernel(inputs, W):
    input_shape = inputs.shape
    x2 = inputs.reshape(-1, EMBEDDING_DIM)
    n_tok = x2.shape[0]
    n_rows = n_tok * SAMPLE_TOKENS

    HALVES = 1
    tok_h = n_tok // HALVES
    rows_h = n_rows // HALVES
    n_blocks = tok_h // BLOCK_T

    wt2 = 2.0 * W.T
    wsq = jnp.sum(W ** 2, axis=1)[None, :]
    colf = jnp.arange(NUM_EMBEDDINGS, dtype=jnp.float32)[None, :]
    table = W.astype(jnp.bfloat16).astype(jnp.float32)

    sc_call = _make_sc_kernel(rows_h)

    def tc_call(h):
        off = h * (tok_h // BLOCK_T)
        return pl.pallas_call(
            functools.partial(_tc_body, n_blocks=n_blocks),
            grid=(n_blocks,),
            in_specs=[
                pl.BlockSpec((BLOCK_T, EMBEDDING_DIM), lambda i: (i + off, 0)),
                pl.BlockSpec((CODE_DIM, NUM_EMBEDDINGS), lambda i: (0, 0)),
                pl.BlockSpec((1, NUM_EMBEDDINGS), lambda i: (0, 0)),
                pl.BlockSpec((1, NUM_EMBEDDINGS), lambda i: (0, 0)),
            ],
            out_specs=[
                pl.BlockSpec((BLOCK_T, SAMPLE_TOKENS), lambda i: (i, 0)),
                pl.BlockSpec((1, 1), lambda i: (0, 0)),
            ],
            out_shape=[
                jax.ShapeDtypeStruct((tok_h, SAMPLE_TOKENS), jnp.int32),
                jax.ShapeDtypeStruct((1, 1), jnp.float32),
            ],
            scratch_shapes=[
                pltpu.SMEM((1, 1), jnp.float32),
            ],
        )(x2, wt2, wsq, colf)

    qs, cnts, sq_total = [], [], None
    for h in range(HALVES):
        idx_h, sq_h = tc_call(h)
        q_h, cnt_h = sc_call(table, idx_h.reshape(-1))
        qs.append(q_h)
        cnts.append(cnt_h)
        sq_total = sq_h if sq_total is None else sq_total + sq_h

    loss = ((1.0 + COMMITMENT_COST) / (n_rows * CODE_DIM)) * sq_total[0, 0]

    cnt = jnp.concatenate(cnts, axis=0)
    perp = pl.pallas_call(
        functools.partial(_perp_body, n_rows=n_rows),
        in_specs=[pl.BlockSpec((HALVES * _NC, NUM_EMBEDDINGS),
                               lambda: (0, 0))],
        out_specs=pl.BlockSpec((1, 1), lambda: (0, 0)),
        out_shape=jax.ShapeDtypeStruct((1, 1), jnp.float32),
    )(cnt)

    q = jnp.concatenate(qs, axis=0)
    return (q.reshape(input_shape), loss, perp[0, 0])
